# Initial kernel scaffold; baseline (speedup 1.0000x reference)
import functools

import jax
import jax.numpy as jnp
from jax import lax
from jax.experimental import pallas as pl
from jax.experimental.pallas import tpu as pltpu
from jax.experimental.pallas import tpu_sc as plsc

BATCH = 16384
HIST = 50
EMBED_DIM = 32

_B = BATCH * HIST
_G = 128
_NW = 32
_GROUPS_PER_W = _B // (_G * _NW)


def _make_gather():
    mesh = plsc.VectorSubcoreMesh(core_axis_name="c", subcore_axis_name="s")

    @functools.partial(
        pl.kernel,
        mesh=mesh,
        out_type=jax.ShapeDtypeStruct((_B, EMBED_DIM), jnp.float32),
        scratch_types=[
            pltpu.VMEM((_GROUPS_PER_W, _G), jnp.int32),
            pltpu.VMEM((_G, EMBED_DIM), jnp.float32),
            pltpu.SemaphoreType.DMA,
        ],
    )
    def gather_kernel(idx_hbm, table_hbm, out_hbm, idx_v, rows_v, sem):
        wid = lax.axis_index("s") * 2 + lax.axis_index("c")
        grp_base = wid * _GROUPS_PER_W
        pltpu.sync_copy(idx_hbm.at[pl.ds(grp_base, _GROUPS_PER_W)], idx_v)

        def body(j):
            pltpu.async_copy(table_hbm.at[idx_v.at[j]], rows_v, sem).wait()
            row0 = (grp_base + j) * _G
            pltpu.sync_copy(rows_v, out_hbm.at[pl.ds(row0, _G)])

        pl.loop(0, _GROUPS_PER_W)(body)

    return gather_kernel


_gather = _make_gather()


def kernel(action, embedding):
    idx = action.reshape(_B // _G, _G).astype(jnp.int32)
    out = _gather(idx, embedding)
    return out.reshape(BATCH, HIST, EMBED_DIM)


# speedup vs baseline: 1.0221x; 1.0221x over previous
import functools

import jax
import jax.numpy as jnp
from jax import lax
from jax.experimental import pallas as pl
from jax.experimental.pallas import tpu as pltpu
from jax.experimental.pallas import tpu_sc as plsc

BATCH = 16384
HIST = 50
EMBED_DIM = 32

_B = BATCH * HIST
_G = 128
_NW = 32
_GROUPS_PER_W = _B // (_G * _NW)


def _make_gather():
    mesh = plsc.VectorSubcoreMesh(core_axis_name="c", subcore_axis_name="s")

    @functools.partial(
        pl.kernel,
        mesh=mesh,
        out_type=jax.ShapeDtypeStruct((_B, EMBED_DIM), jnp.float32),
        scratch_types=[
            pltpu.VMEM((_GROUPS_PER_W, _G), jnp.int32),
            pltpu.VMEM((_G, EMBED_DIM), jnp.float32),
            pltpu.SemaphoreType.DMA,
        ],
        compiler_params=pltpu.CompilerParams(use_tc_tiling_on_sc=False),
    )
    def gather_kernel(idx_hbm, table_hbm, out_hbm, idx_v, rows_v, sem):
        wid = lax.axis_index("s") * 2 + lax.axis_index("c")
        grp_base = wid * _GROUPS_PER_W
        pltpu.sync_copy(idx_hbm.at[pl.ds(grp_base, _GROUPS_PER_W)], idx_v)

        def body(j):
            pltpu.async_copy(table_hbm.at[idx_v.at[j]], rows_v, sem).wait()
            row0 = (grp_base + j) * _G
            pltpu.sync_copy(rows_v, out_hbm.at[pl.ds(row0, _G)])

        pl.loop(0, _GROUPS_PER_W)(body)

    return gather_kernel


_gather = _make_gather()


def kernel(action, embedding):
    idx = action.reshape(_B // _G, _G).astype(jnp.int32)
    out = _gather(idx, embedding)
    return out.reshape(BATCH, HIST, EMBED_DIM)


# speedup vs baseline: 1.1127x; 1.0887x over previous
import functools

import jax
import jax.numpy as jnp
from jax import lax
from jax.experimental import pallas as pl
from jax.experimental.pallas import tpu as pltpu
from jax.experimental.pallas import tpu_sc as plsc

BATCH = 16384
HIST = 50
EMBED_DIM = 32

_B = BATCH * HIST
_G = 128
_NW = 32
_GROUPS_PER_W = _B // (_G * _NW)
_K = 10
_SG = _GROUPS_PER_W // _K


def _make_gather():
    mesh = plsc.VectorSubcoreMesh(core_axis_name="c", subcore_axis_name="s")

    @functools.partial(
        pl.kernel,
        mesh=mesh,
        out_type=jax.ShapeDtypeStruct((_B, EMBED_DIM), jnp.float32),
        scratch_types=[
            pltpu.VMEM((_GROUPS_PER_W, _G), jnp.int32),
            pltpu.VMEM((_K * _G, EMBED_DIM), jnp.float32),
            pltpu.VMEM((_K * _G, EMBED_DIM), jnp.float32),
            pltpu.SemaphoreType.DMA,
            pltpu.SemaphoreType.DMA,
            pltpu.SemaphoreType.DMA,
            pltpu.SemaphoreType.DMA,
        ],
        compiler_params=pltpu.CompilerParams(use_tc_tiling_on_sc=False),
    )
    def gather_kernel(idx_hbm, table_hbm, out_hbm, idx_v,
                      rows0, rows1, gs0, gs1, os0, os1):
        rows = (rows0, rows1)
        gsem = (gs0, gs1)
        osem = (os0, os1)
        wid = lax.axis_index("s") * 2 + lax.axis_index("c")
        grp_base = wid * _GROUPS_PER_W
        pltpu.sync_copy(idx_hbm.at[pl.ds(grp_base, _GROUPS_PER_W)], idx_v)

        def fire_gathers(sg, b):
            for k in range(_K):
                pltpu.async_copy(table_hbm.at[idx_v.at[sg * _K + k]],
                                 rows[b].at[pl.ds(k * _G, _G)], gsem[b])

        def drain_gathers(b):
            pltpu.make_async_copy(out_hbm.at[pl.ds(0, _K * _G)],
                                  rows[b], gsem[b]).wait()

        def fire_out(sg, b):
            row0 = (grp_base + sg * _K) * _G
            pltpu.async_copy(rows[b], out_hbm.at[pl.ds(row0, _K * _G)],
                             osem[b])

        def wait_out(b):
            pltpu.make_async_copy(rows[b], out_hbm.at[pl.ds(0, _K * _G)],
                                  osem[b]).wait()

        fire_gathers(0, 0)
        fire_gathers(1, 1)

        def body(sg0):
            for b in range(2):
                sg = sg0 + b
                drain_gathers(b)
                fire_out(sg, b)

                @pl.when(sg + 2 < _SG)
                def _():
                    wait_out(b)
                    fire_gathers(sg + 2, b)

        pl.loop(0, _SG, step=2)(body)
        wait_out(0)
        wait_out(1)

    return gather_kernel


_gather = _make_gather()


def kernel(action, embedding):
    idx = action.reshape(_B // _G, _G).astype(jnp.int32)
    out = _gather(idx, embedding)
    return out.reshape(BATCH, HIST, EMBED_DIM)


# speedup vs baseline: 1.5401x; 1.3841x over previous
import functools

import jax
import jax.numpy as jnp
from jax import lax
from jax.experimental import pallas as pl
from jax.experimental.pallas import tpu as pltpu
from jax.experimental.pallas import tpu_sc as plsc

BATCH = 16384
HIST = 50
EMBED_DIM = 32

_G = 128
_NBLK = HIST * (BATCH // _G)
_NW = 32
_BPW = _NBLK // _NW
_TPH = BATCH // _G


def _make_gather():
    mesh = plsc.VectorSubcoreMesh(core_axis_name="c", subcore_axis_name="s")

    @functools.partial(
        pl.kernel,
        mesh=mesh,
        out_type=jax.ShapeDtypeStruct((HIST * 4 * _TPH, 8, _G), jnp.float32),
        scratch_types=[
            pltpu.VMEM((_BPW, _G), jnp.int32),
            pltpu.VMEM((_G, EMBED_DIM), jnp.float32),
            pltpu.VMEM((_G, EMBED_DIM), jnp.float32),
            pltpu.VMEM((4, 8, _G), jnp.float32),
            pltpu.VMEM((4, 8, _G), jnp.float32),
            pltpu.SemaphoreType.DMA,
            pltpu.SemaphoreType.DMA,
            pltpu.SemaphoreType.DMA,
            pltpu.SemaphoreType.DMA,
        ],
        compiler_params=pltpu.CompilerParams(use_tc_tiling_on_sc=False,
                                             needs_layout_passes=False),
    )
    def gather_kernel(idx_hbm, table_hbm, out_hbm, idx_v,
                      rows0, rows1, t0, t1, gs0, gs1, os0, os1):
        rows = (rows0, rows1)
        tile = (t0, t1)
        gsem = (gs0, gs1)
        osem = (os0, os1)
        wid = lax.axis_index("s") * 2 + lax.axis_index("c")
        blk_base = wid * _BPW
        pltpu.sync_copy(idx_hbm.at[pl.ds(blk_base, _BPW)], idx_v)

        lane = lax.iota(jnp.int32, 16)
        row_idx = [lane + c * 16 for c in range(8)]

        def fire_gather(i, rb):
            pltpu.async_copy(table_hbm.at[idx_v.at[i]], rows[rb], gsem[rb])

        def drain_gather(rb):
            pltpu.make_async_copy(table_hbm.at[pl.ds(0, _G)],
                                  rows[rb], gsem[rb]).wait()

        def transpose(rb, tb):
            for d in range(EMBED_DIM):
                col = jnp.full((16,), d, jnp.int32)
                for c in range(8):
                    v = plsc.load_gather(rows[rb], [row_idx[c], col])
                    tile[tb][d // 8, d % 8, pl.ds(c * 16, 16)] = v

        def fire_out(i, tb):
            j = blk_base + i
            row0 = (j // _TPH) * 512 + (j % _TPH)
            for dt in range(4):
                pltpu.async_copy(tile[tb].at[dt],
                                 out_hbm.at[row0 + dt * _TPH], osem[tb])

        def wait_out(tb):
            pltpu.make_async_copy(tile[tb], out_hbm.at[pl.ds(0, 4)],
                                  osem[tb]).wait()

        fire_gather(0, 0)

        def body(i0):
            for b in range(2):
                i = i0 + b

                @pl.when(i + 1 < _BPW)
                def _():
                    fire_gather(i + 1, 1 - b)

                drain_gather(b)

                @pl.when(i >= 2)
                def _():
                    wait_out(b)

                transpose(b, b)
                fire_out(i, b)

        pl.loop(0, _BPW, step=2)(body)
        wait_out(0)
        wait_out(1)

    return gather_kernel


_gather = _make_gather()


def kernel(action, embedding):
    idx = action.T.reshape(_NBLK, _G)
    out5 = _gather(idx, embedding)
    y = out5.reshape(HIST, 4, _TPH, 8, _G).transpose(2, 4, 0, 1, 3)
    return y.reshape(BATCH, HIST, EMBED_DIM)


# speedup vs baseline: 1.5409x; 1.0005x over previous
import functools

import jax
import jax.numpy as jnp
from jax import lax
from jax.experimental import pallas as pl
from jax.experimental.pallas import tpu as pltpu
from jax.experimental.pallas import tpu_sc as plsc

BATCH = 16384
HIST = 50
EMBED_DIM = 32

_G = 128
_NBLK = HIST * (BATCH // _G)
_NW = 32
_BPW = _NBLK // _NW
_TPH = BATCH // _G


def _make_gather():
    mesh = plsc.VectorSubcoreMesh(core_axis_name="c", subcore_axis_name="s")

    @functools.partial(
        pl.kernel,
        mesh=mesh,
        out_type=jax.ShapeDtypeStruct((HIST * 4 * _TPH, 8, _G), jnp.float32),
        scratch_types=[
            pltpu.VMEM((_BPW, _G), jnp.int32),
            pltpu.VMEM((_G, EMBED_DIM), jnp.float32),
            pltpu.VMEM((_G, EMBED_DIM), jnp.float32),
            pltpu.VMEM((4, 8, _G), jnp.float32),
            pltpu.VMEM((4, 8, _G), jnp.float32),
            pltpu.SemaphoreType.DMA,
            pltpu.SemaphoreType.DMA,
            pltpu.SemaphoreType.DMA,
            pltpu.SemaphoreType.DMA,
        ],
        compiler_params=pltpu.CompilerParams(use_tc_tiling_on_sc=False,
                                             needs_layout_passes=False,
                                             disable_bounds_checks=True),
    )
    def gather_kernel(idx_hbm, table_hbm, out_hbm, idx_v,
                      rows0, rows1, t0, t1, gs0, gs1, os0, os1):
        rows = (rows0, rows1)
        tile = (t0, t1)
        gsem = (gs0, gs1)
        osem = (os0, os1)
        wid = lax.axis_index("s") * 2 + lax.axis_index("c")
        blk_base = wid * _BPW
        pltpu.sync_copy(idx_hbm.at[pl.ds(blk_base, _BPW)], idx_v)

        lane = lax.iota(jnp.int32, 16)
        row_idx = [lane + c * 16 for c in range(8)]

        def fire_gather(i, rb):
            pltpu.async_copy(table_hbm.at[idx_v.at[i]], rows[rb], gsem[rb])

        def drain_gather(rb):
            pltpu.make_async_copy(table_hbm.at[pl.ds(0, _G)],
                                  rows[rb], gsem[rb]).wait()

        def transpose(rb, tb):
            for d in range(EMBED_DIM):
                col = jnp.full((16,), d, jnp.int32)
                for c in range(8):
                    v = plsc.load_gather(rows[rb], [row_idx[c], col])
                    tile[tb][d // 8, d % 8, pl.ds(c * 16, 16)] = v

        def fire_out(i, tb):
            j = blk_base + i
            row0 = (j // _TPH) * 512 + (j % _TPH)
            for dt in range(4):
                pltpu.async_copy(tile[tb].at[dt],
                                 out_hbm.at[row0 + dt * _TPH], osem[tb])

        def wait_out(tb):
            pltpu.make_async_copy(tile[tb], out_hbm.at[pl.ds(0, 4)],
                                  osem[tb]).wait()

        fire_gather(0, 0)

        def body(i0):
            for b in range(2):
                i = i0 + b

                @pl.when(i + 1 < _BPW)
                def _():
                    fire_gather(i + 1, 1 - b)

                drain_gather(b)

                @pl.when(i >= 2)
                def _():
                    wait_out(b)

                transpose(b, b)
                fire_out(i, b)

        pl.loop(0, _BPW, step=2)(body)
        wait_out(0)
        wait_out(1)

    return gather_kernel


_gather = _make_gather()


def kernel(action, embedding):
    idx = action.T.reshape(_NBLK, _G)
    out5 = _gather(idx, embedding)
    y = out5.reshape(HIST, 4, _TPH, 8, _G).transpose(2, 4, 0, 1, 3)
    return y.reshape(BATCH, HIST, EMBED_DIM)


# speedup vs baseline: 1.8340x; 1.1902x over previous
import functools

import jax
import jax.numpy as jnp
from jax import lax
from jax.experimental import pallas as pl
from jax.experimental.pallas import tpu as pltpu
from jax.experimental.pallas import tpu_sc as plsc

BATCH = 16384
HIST = 50
EMBED_DIM = 32

_G = 128
_NBLK = HIST * (BATCH // _G)
_NW = 32
_BPW = _NBLK // _NW
_TPH = BATCH // _G


def _make_gather():
    mesh = plsc.VectorSubcoreMesh(core_axis_name="c", subcore_axis_name="s")

    @functools.partial(
        pl.kernel,
        mesh=mesh,
        out_type=jax.ShapeDtypeStruct((HIST * 4 * _TPH, 8, _G), jnp.float32),
        scratch_types=[
            pltpu.VMEM((_BPW, _G), jnp.int32),
            pltpu.VMEM((_G, EMBED_DIM), jnp.float32),
            pltpu.VMEM((_G, EMBED_DIM), jnp.float32),
            pltpu.VMEM((EMBED_DIM, _G), jnp.float32),
            pltpu.VMEM((EMBED_DIM, _G), jnp.float32),
            pltpu.SemaphoreType.DMA,
            pltpu.SemaphoreType.DMA,
            pltpu.SemaphoreType.DMA,
            pltpu.SemaphoreType.DMA,
        ],
        compiler_params=pltpu.CompilerParams(use_tc_tiling_on_sc=False,
                                             needs_layout_passes=False,
                                             disable_bounds_checks=True),
    )
    def gather_kernel(idx_hbm, table_hbm, out_hbm, idx_v,
                      rows0, rows1, t0, t1, gs0, gs1, os0, os1):
        rows = (rows0, rows1)
        tile = (t0, t1)
        gsem = (gs0, gs1)
        osem = (os0, os1)
        wid = lax.axis_index("s") * 2 + lax.axis_index("c")
        blk_base = wid * _BPW
        pltpu.sync_copy(idx_hbm.at[pl.ds(blk_base, _BPW)], idx_v)

        lane = lax.iota(jnp.int32, 16)
        dvec = [lane + h * 16 for h in range(2)]

        def fire_gather(i, rb):
            pltpu.async_copy(table_hbm.at[idx_v.at[i]], rows[rb], gsem[rb])

        def drain_gather(rb):
            pltpu.make_async_copy(table_hbm.at[pl.ds(0, _G)],
                                  rows[rb], gsem[rb]).wait()

        def transpose(rb, tb):
            for b1 in range(_G):
                bvec = jnp.full((16,), b1, jnp.int32)
                for h in range(2):
                    v = rows[rb][b1, pl.ds(h * 16, 16)]
                    plsc.store_scatter(tile[tb], [dvec[h], bvec], v)

        def fire_out(i, tb):
            j = blk_base + i
            row0 = (j // _TPH) * 512 + (j % _TPH)
            for dt in range(4):
                pltpu.async_copy(tile[tb].at[pl.ds(dt * 8, 8)],
                                 out_hbm.at[row0 + dt * _TPH], osem[tb])

        def wait_out(tb):
            for dt in range(4):
                pltpu.make_async_copy(tile[tb].at[pl.ds(dt * 8, 8)],
                                      out_hbm.at[0], osem[tb]).wait()

        fire_gather(0, 0)

        def body(i0):
            for b in range(2):
                i = i0 + b

                @pl.when(i + 1 < _BPW)
                def _():
                    fire_gather(i + 1, 1 - b)

                drain_gather(b)

                @pl.when(i >= 2)
                def _():
                    wait_out(b)

                transpose(b, b)
                fire_out(i, b)

        pl.loop(0, _BPW, step=2)(body)
        wait_out(0)
        wait_out(1)

    return gather_kernel


_gather = _make_gather()


def kernel(action, embedding):
    idx = action.T.reshape(_NBLK, _G)
    out5 = _gather(idx, embedding)
    y = out5.reshape(HIST, 4, _TPH, 8, _G).transpose(2, 4, 0, 1, 3)
    return y.reshape(BATCH, HIST, EMBED_DIM)
